# baseline (speedup 1.0000x reference)
import functools

import jax
import jax.numpy as jnp
from jax import lax
from jax.experimental import pallas as pl
from jax.experimental.pallas import tpu as pltpu
from jax.experimental.pallas import tpu_sc as plsc

K = 16
CHUNK = 128


def _gmf_sc(B, NC, NS):
    NW = NC * NS
    b_per_w = B // NW
    n_chunks = b_per_w // CHUNK
    n_blocks = b_per_w // K
    mesh = plsc.VectorSubcoreMesh(core_axis_name="c", subcore_axis_name="s")

    @functools.partial(
        pl.kernel,
        mesh=mesh,
        out_type=jax.ShapeDtypeStruct((B,), jnp.float32),
        compiler_params=pltpu.CompilerParams(
            needs_layout_passes=False, use_tc_tiling_on_sc=False),
        scratch_types=[
            pltpu.VMEM((n_chunks, CHUNK), jnp.int32),
            pltpu.VMEM((n_chunks, CHUNK), jnp.int32),
            pltpu.VMEM((b_per_w, K), jnp.float32),
            pltpu.VMEM((b_per_w, K), jnp.float32),
            pltpu.VMEM((K,), jnp.float32),
            pltpu.VMEM((K,), jnp.float32),
            pltpu.VMEM((b_per_w,), jnp.float32),
            pltpu.SemaphoreType.DMA,
        ],
    )
    def gmf(user_hbm, item_hbm, ut_hbm, it_hbm, w_hbm, b_hbm, out_hbm,
            uidx, iidx, urows, irows, wv, bv, outv, sem):
        wid = lax.axis_index("s") * NC + lax.axis_index("c")
        base_chunk = wid * n_chunks

        pltpu.sync_copy(w_hbm, wv)
        pltpu.sync_copy(b_hbm, bv)
        for c in range(n_chunks):
            pltpu.sync_copy(user_hbm.at[base_chunk + c], uidx.at[c])
            pltpu.sync_copy(item_hbm.at[base_chunk + c], iidx.at[c])

        copies = []
        for c in range(n_chunks):
            copies.append(pltpu.async_copy(
                ut_hbm.at[uidx.at[c]], urows.at[pl.ds(c * CHUNK, CHUNK)], sem))
            copies.append(pltpu.async_copy(
                it_hbm.at[iidx.at[c]], irows.at[pl.ds(c * CHUNK, CHUNK)], sem))
        for cp in copies:
            cp.wait()

        wvec = wv[...]
        bias = bv[...]
        lane = lax.iota(jnp.int32, K)
        masks = [lane == j for j in range(K)]

        def block_body(blk, carry):
            rb = blk * K
            acc = jnp.zeros((K,), jnp.float32)
            for j in range(K):
                u = urows[rb + j]
                v = irows[rb + j]
                s = jnp.sum(u * v * wvec)
                acc = jnp.where(masks[j], s, acc)
            outv[pl.ds(rb, K)] = 1.0 / (1.0 + jnp.exp(-(acc + bias)))
            return carry

        lax.fori_loop(0, n_blocks, block_body, 0)
        pltpu.sync_copy(outv, out_hbm.at[pl.ds(wid * b_per_w, b_per_w)])

    return gmf


def kernel(user, item, user_table, item_table, W, b):
    B = user.shape[0]
    info = plsc.get_sparse_core_info()
    NC, NS = info.num_cores, info.num_subcores
    NW = NC * NS
    n_chunks_total = B // CHUNK

    user_i = user.astype(jnp.int32).reshape(n_chunks_total, CHUNK)
    item_i = item.astype(jnp.int32).reshape(n_chunks_total, CHUNK)
    w_vec = W.reshape(K)
    b_splat = jnp.broadcast_to(b.reshape(1), (K,))

    out = _gmf_sc(B, NC, NS)(user_i, item_i, user_table, item_table,
                             w_vec, b_splat)
    return out.reshape(B, 1)


# speedup vs baseline: 1.4779x; 1.4779x over previous
import functools

import jax
import jax.numpy as jnp
from jax import lax
from jax.experimental import pallas as pl
from jax.experimental.pallas import tpu as pltpu
from jax.experimental.pallas import tpu_sc as plsc

K = 16


def _gmf_sc(B, NC, NS):
    NW = NC * NS
    b_per_w = B // NW
    n_blocks = b_per_w // K
    mesh = plsc.VectorSubcoreMesh(core_axis_name="c", subcore_axis_name="s")

    @functools.partial(
        pl.kernel,
        mesh=mesh,
        out_type=jax.ShapeDtypeStruct((B,), jnp.float32),
        compiler_params=pltpu.CompilerParams(
            needs_layout_passes=False, use_tc_tiling_on_sc=True),
        scratch_types=[
            pltpu.VMEM((b_per_w,), jnp.int32),
            pltpu.VMEM((b_per_w,), jnp.int32),
            pltpu.VMEM((2, K, K), jnp.float32),
            pltpu.VMEM((2, K, K), jnp.float32),
            pltpu.VMEM((K,), jnp.float32),
            pltpu.VMEM((K,), jnp.float32),
            pltpu.VMEM((b_per_w,), jnp.float32),
            pltpu.SemaphoreType.DMA,
        ],
    )
    def gmf(user_hbm, item_hbm, ut_hbm, it_hbm, w_hbm, b_hbm, out_hbm,
            uidxv, iidxv, ubuf, ibuf, wv, bv, outv, sem):
        wid = lax.axis_index("s") * NC + lax.axis_index("c")
        base = wid * b_per_w

        pltpu.sync_copy(w_hbm, wv)
        pltpu.sync_copy(b_hbm, bv)
        pltpu.sync_copy(user_hbm.at[pl.ds(base, b_per_w)], uidxv)
        pltpu.sync_copy(item_hbm.at[pl.ds(base, b_per_w)], iidxv)

        wvec = wv[...]
        bias = bv[...]
        lane = lax.iota(jnp.int32, K)
        masks = [lane == j for j in range(K)]
        izero = jnp.zeros((K,), jnp.int32)

        def issue(blk, slot):
            uv = uidxv[pl.ds(blk * K, K)]
            iv = iidxv[pl.ds(blk * K, K)]
            for j in range(K):
                ui = jnp.sum(jnp.where(masks[j], uv, izero))
                ii = jnp.sum(jnp.where(masks[j], iv, izero))
                pltpu.async_copy(ut_hbm.at[ui], ubuf.at[slot, j], sem)
                pltpu.async_copy(it_hbm.at[ii], ibuf.at[slot, j], sem)

        def drain():
            for j in range(K):
                pltpu.make_async_copy(ut_hbm.at[0], ubuf.at[0, j], sem).wait()
                pltpu.make_async_copy(it_hbm.at[0], ibuf.at[0, j], sem).wait()

        def compute(blk, slot):
            acc = jnp.zeros((K,), jnp.float32)
            for j in range(K):
                u = ubuf[slot, j]
                v = ibuf[slot, j]
                s = jnp.sum(u * v * wvec)
                acc = jnp.where(masks[j], s, acc)
            outv[pl.ds(blk * K, K)] = 1.0 / (1.0 + jnp.exp(-(acc + bias)))

        issue(0, 0)

        def body(t, carry):
            blk = t * 2
            issue(blk + 1, 1)
            drain()
            compute(blk, 0)

            @pl.when(t < n_blocks // 2 - 1)
            def _():
                issue(blk + 2, 0)

            drain()
            compute(blk + 1, 1)
            return carry

        lax.fori_loop(0, n_blocks // 2, body, 0)
        pltpu.sync_copy(outv, out_hbm.at[pl.ds(base, b_per_w)])

    return gmf


def kernel(user, item, user_table, item_table, W, b):
    B = user.shape[0]
    info = plsc.get_sparse_core_info()
    NC, NS = info.num_cores, info.num_subcores

    user_i = user.astype(jnp.int32)
    item_i = item.astype(jnp.int32)
    w_vec = W.reshape(K)
    b_splat = jnp.broadcast_to(b.reshape(1), (K,))

    out = _gmf_sc(B, NC, NS)(user_i, item_i, user_table, item_table,
                             w_vec, b_splat)
    return out.reshape(B, 1)


# speedup vs baseline: 1.4881x; 1.0069x over previous
import functools

import jax
import jax.numpy as jnp
from jax import lax
from jax.experimental import pallas as pl
from jax.experimental.pallas import tpu as pltpu
from jax.experimental.pallas import tpu_sc as plsc

K = 16


def _gmf_sc(B, NC, NS):
    NW = NC * NS
    b_per_w = B // NW
    n_blocks = b_per_w // K
    mesh = plsc.VectorSubcoreMesh(core_axis_name="c", subcore_axis_name="s")

    @functools.partial(
        pl.kernel,
        mesh=mesh,
        out_type=jax.ShapeDtypeStruct((B,), jnp.float32),
        compiler_params=pltpu.CompilerParams(
            needs_layout_passes=False, use_tc_tiling_on_sc=True),
        scratch_types=[
            pltpu.VMEM((b_per_w,), jnp.int32),
            pltpu.VMEM((b_per_w,), jnp.int32),
            pltpu.VMEM((b_per_w // 2, K), jnp.float32),
            pltpu.VMEM((b_per_w // 2, K), jnp.float32),
            pltpu.VMEM((K,), jnp.float32),
            pltpu.VMEM((K,), jnp.float32),
            pltpu.VMEM((b_per_w,), jnp.float32),
            pltpu.SemaphoreType.DMA,
        ],
    )
    def gmf(user_hbm, item_hbm, ut_hbm, it_hbm, w_hbm, b_hbm, out_hbm,
            uidxv, iidxv, ubuf, ibuf, wv, bv, outv, sem):
        wid = lax.axis_index("s") * NC + lax.axis_index("c")
        base = wid * b_per_w

        pltpu.sync_copy(w_hbm, wv)
        pltpu.sync_copy(b_hbm, bv)
        pltpu.sync_copy(user_hbm.at[pl.ds(base, b_per_w)], uidxv)
        pltpu.sync_copy(item_hbm.at[pl.ds(base, b_per_w)], iidxv)

        wvec = wv[...]
        bias = bv[...]
        lane = lax.iota(jnp.int32, K)
        masks = [lane == j for j in range(K)]
        izero = jnp.zeros((K,), jnp.int32)

        half_blocks = n_blocks // 2

        for half in range(2):
            hoff = half * half_blocks * K

            def issue_body(blk, carry):
                uv = uidxv[pl.ds(hoff + blk * K, K)]
                iv = iidxv[pl.ds(hoff + blk * K, K)]
                for j in range(K):
                    ui = jnp.sum(jnp.where(masks[j], uv, izero))
                    ii = jnp.sum(jnp.where(masks[j], iv, izero))
                    row = blk * K + j
                    pltpu.async_copy(ut_hbm.at[ui], ubuf.at[row], sem)
                    pltpu.async_copy(it_hbm.at[ii], ibuf.at[row], sem)
                return carry

            def drain_body(blk, carry):
                for j in range(K):
                    pltpu.make_async_copy(ut_hbm.at[0], ubuf.at[j], sem).wait()
                    pltpu.make_async_copy(it_hbm.at[0], ibuf.at[j], sem).wait()
                return carry

            def compute_body(blk, carry):
                acc = jnp.zeros((K,), jnp.float32)
                for j in range(K):
                    row = blk * K + j
                    u = ubuf[row]
                    v = ibuf[row]
                    s = jnp.sum(u * v * wvec)
                    acc = jnp.where(masks[j], s, acc)
                outv[pl.ds(hoff + blk * K, K)] = (
                    1.0 / (1.0 + jnp.exp(-(acc + bias))))
                return carry

            lax.fori_loop(0, half_blocks, issue_body, 0)
            lax.fori_loop(0, half_blocks, drain_body, 0)
            lax.fori_loop(0, half_blocks, compute_body, 0)
        pltpu.sync_copy(outv, out_hbm.at[pl.ds(base, b_per_w)])

    return gmf


def kernel(user, item, user_table, item_table, W, b):
    B = user.shape[0]
    info = plsc.get_sparse_core_info()
    NC, NS = info.num_cores, info.num_subcores

    user_i = user.astype(jnp.int32)
    item_i = item.astype(jnp.int32)
    w_vec = W.reshape(K)
    b_splat = jnp.broadcast_to(b.reshape(1), (K,))

    out = _gmf_sc(B, NC, NS)(user_i, item_i, user_table, item_table,
                             w_vec, b_splat)
    return out.reshape(B, 1)


# speedup vs baseline: 1.4954x; 1.0049x over previous
import functools

import jax
import jax.numpy as jnp
from jax import lax
from jax.experimental import pallas as pl
from jax.experimental.pallas import tpu as pltpu
from jax.experimental.pallas import tpu_sc as plsc

K = 16


def _gmf_sc(B, NC, NS):
    NW = NC * NS
    b_per_w = B // NW
    n_blocks = b_per_w // K
    mesh = plsc.VectorSubcoreMesh(core_axis_name="c", subcore_axis_name="s")

    @functools.partial(
        pl.kernel,
        mesh=mesh,
        out_type=jax.ShapeDtypeStruct((B,), jnp.float32),
        compiler_params=pltpu.CompilerParams(
            needs_layout_passes=False, use_tc_tiling_on_sc=True),
        scratch_types=[
            pltpu.VMEM((b_per_w,), jnp.int32),
            pltpu.VMEM((b_per_w,), jnp.int32),
            pltpu.VMEM((b_per_w // 2, K), jnp.float32),
            pltpu.VMEM((b_per_w // 2, K), jnp.float32),
            pltpu.VMEM((K,), jnp.float32),
            pltpu.VMEM((K,), jnp.float32),
            pltpu.VMEM((b_per_w,), jnp.float32),
            pltpu.SemaphoreType.DMA,
        ],
    )
    def gmf(user_hbm, item_hbm, ut_hbm, it_hbm, w_hbm, b_hbm, out_hbm,
            uidxv, iidxv, ubuf, ibuf, wv, bv, outv, sem):
        wid = lax.axis_index("s") * NC + lax.axis_index("c")
        base = wid * b_per_w

        pltpu.sync_copy(w_hbm, wv)
        pltpu.sync_copy(b_hbm, bv)
        pltpu.sync_copy(user_hbm.at[pl.ds(base, b_per_w)], uidxv)
        pltpu.sync_copy(item_hbm.at[pl.ds(base, b_per_w)], iidxv)

        wvec = wv[...]
        bias = bv[...]
        lane = lax.iota(jnp.int32, K)
        masks = [lane == j for j in range(K)]
        izero = jnp.zeros((K,), jnp.int32)

        half_rows = b_per_w // 2
        half_blocks = n_blocks // 2

        for half in range(2):
            hoff = half * half_blocks * K

            def issue_body(blk, carry):
                uv = uidxv[pl.ds(hoff + blk * K, K)]
                iv = iidxv[pl.ds(hoff + blk * K, K)]
                for j in range(K):
                    row = blk * K + j
                    ui = jnp.sum(jnp.where(masks[j], uv, izero))
                    ii = jnp.sum(jnp.where(masks[j], iv, izero))
                    pltpu.async_copy(ut_hbm.at[ui], ubuf.at[row], sem)
                    pltpu.async_copy(it_hbm.at[ii], ibuf.at[row], sem)
                return carry

            def compute_body(blk, carry):
                acc = jnp.zeros((K,), jnp.float32)
                for j in range(K):
                    row = blk * K + j
                    u = ubuf[row]
                    v = ibuf[row]
                    s = jnp.sum(u * v * wvec)
                    acc = jnp.where(masks[j], s, acc)
                outv[pl.ds(hoff + blk * K, K)] = (
                    1.0 / (1.0 + jnp.exp(-(acc + bias))))
                return carry

            lax.fori_loop(0, half_blocks, issue_body, 0)
            pltpu.make_async_copy(
                ut_hbm.at[pl.ds(0, half_rows)], ubuf, sem).wait()
            pltpu.make_async_copy(
                it_hbm.at[pl.ds(0, half_rows)], ibuf, sem).wait()
            lax.fori_loop(0, half_blocks, compute_body, 0)
        pltpu.sync_copy(outv, out_hbm.at[pl.ds(base, b_per_w)])

    return gmf


def kernel(user, item, user_table, item_table, W, b):
    B = user.shape[0]
    info = plsc.get_sparse_core_info()
    NC, NS = info.num_cores, info.num_subcores

    user_i = user.astype(jnp.int32)
    item_i = item.astype(jnp.int32)
    w_vec = W.reshape(K)
    b_splat = jnp.broadcast_to(b.reshape(1), (K,))

    out = _gmf_sc(B, NC, NS)(user_i, item_i, user_table, item_table,
                             w_vec, b_splat)
    return out.reshape(B, 1)
